# Initial kernel scaffold; baseline (speedup 1.0000x reference)
import jax
import jax.numpy as jnp
from jax.experimental import pallas as pl


def kernel(input, weight):
    raise NotImplementedError("write your pallas kernel here")


# speedup vs baseline: 1.7998x; 1.7998x over previous
import functools

import jax
import jax.numpy as jnp
from jax import lax
from jax.experimental import pallas as pl
from jax.experimental.pallas import tpu as pltpu
from jax.experimental.pallas import tpu_sc as plsc

_NC = 2
_NS = 16
_NW = _NC * _NS
_CH = 128


def _build_gather(V, D, N):
    assert N % (_NW * _CH) == 0
    cpw = N // (_NW * _CH)
    assert cpw % 2 == 0
    mesh = plsc.VectorSubcoreMesh(core_axis_name="c", subcore_axis_name="s")

    @functools.partial(
        pl.kernel,
        mesh=mesh,
        out_type=jax.ShapeDtypeStruct((N, D), jnp.float32),
        compiler_params=pltpu.CompilerParams(use_tc_tiling_on_sc=False),
        scratch_types=[
            pltpu.VMEM((cpw, _CH), jnp.int32),
            pltpu.VMEM((_CH, D), jnp.float32),
            pltpu.VMEM((_CH, D), jnp.float32),
            pltpu.SemaphoreType.DMA,
            pltpu.SemaphoreType.DMA,
            pltpu.SemaphoreType.DMA,
            pltpu.SemaphoreType.DMA,
        ],
    )
    def gather_kernel(table_hbm, idx_hbm, out_hbm, idx_v, buf0, buf1,
                      g0, g1, s0, s1):
        wid = lax.axis_index("s") * _NC + lax.axis_index("c")
        cbase = wid * cpw
        pltpu.sync_copy(idx_hbm.at[pl.ds(cbase, cpw)], idx_v)

        bufs = (buf0, buf1)
        gsems = (g0, g1)
        ssems = (s0, s1)

        def step(g, carry):
            hs = []
            for b in range(2):
                i = 2 * g + b
                hs.append(pltpu.async_copy(
                    table_hbm.at[idx_v.at[i]], bufs[b], gsems[b]))
            outs = []
            for b in range(2):
                i = 2 * g + b
                hs[b].wait()
                outs.append(pltpu.async_copy(
                    bufs[b], out_hbm.at[pl.ds((cbase + i) * _CH, _CH)],
                    ssems[b]))
            for b in range(2):
                outs[b].wait()
            return carry

        lax.fori_loop(0, cpw // 2, step, 0)

    return gather_kernel


def kernel(input, weight):
    B, H = input.shape
    V, D = weight.shape
    N = B * H
    idx = input.reshape(N // _CH, _CH).astype(jnp.int32)
    out = _build_gather(V, D, N)(weight, idx)
    return out.reshape(B, H, D)


# speedup vs baseline: 1.8732x; 1.0408x over previous
import functools

import jax
import jax.numpy as jnp
from jax import lax
from jax.experimental import pallas as pl
from jax.experimental.pallas import tpu as pltpu
from jax.experimental.pallas import tpu_sc as plsc

_NC = 2
_NS = 16
_NW = _NC * _NS
_CH = 128
_NBUF = 4


def _build_gather(V, D, N):
    assert N % (_NW * _CH) == 0
    cpw = N // (_NW * _CH)
    ngrp = cpw // _NBUF
    assert ngrp % 2 == 0 and ngrp >= 4
    mesh = plsc.VectorSubcoreMesh(core_axis_name="c", subcore_axis_name="s")

    @functools.partial(
        pl.kernel,
        mesh=mesh,
        out_type=jax.ShapeDtypeStruct((N, D), jnp.float32),
        compiler_params=pltpu.CompilerParams(use_tc_tiling_on_sc=False),
        scratch_types=(
            [pltpu.VMEM((cpw, _CH), jnp.int32)]
            + [pltpu.VMEM((_CH, D), jnp.float32) for _ in range(2 * _NBUF)]
            + [pltpu.SemaphoreType.DMA for _ in range(4 * _NBUF)]
        ),
    )
    def gather_kernel(table_hbm, idx_hbm, out_hbm, idx_v, *rest):
        bufs = (rest[0:_NBUF], rest[_NBUF:2 * _NBUF])
        sems = rest[2 * _NBUF:]
        gsems = (sems[0:_NBUF], sems[_NBUF:2 * _NBUF])
        ssems = (sems[2 * _NBUF:3 * _NBUF], sems[3 * _NBUF:4 * _NBUF])

        wid = lax.axis_index("s") * _NC + lax.axis_index("c")
        cbase = wid * cpw
        pltpu.sync_copy(idx_hbm.at[pl.ds(cbase, cpw)], idx_v)

        def g_copy(i, bank, b):
            return pltpu.make_async_copy(
                table_hbm.at[idx_v.at[i]], bufs[bank][b], gsems[bank][b])

        def s_copy(i, bank, b):
            return pltpu.make_async_copy(
                bufs[bank][b], out_hbm.at[pl.ds((cbase + i) * _CH, _CH)],
                ssems[bank][b])

        for bank in range(2):
            for b in range(_NBUF):
                g_copy(bank * _NBUF + b, bank, b).start()

        def step(h, carry):
            for bank in range(2):
                j = 2 * h + bank
                for b in range(_NBUF):
                    i = j * _NBUF + b
                    g_copy(i, bank, b).wait()
                    s_copy(i, bank, b).start()
            for bank in range(2):
                for b in range(_NBUF):
                    s_copy((2 * h + bank) * _NBUF + b, bank, b).wait()
                    g_copy((2 * h + 2 + bank) * _NBUF + b, bank, b).start()
            return carry

        lax.fori_loop(0, ngrp // 2 - 1, step, 0)

        for bank in range(2):
            j = ngrp - 2 + bank
            for b in range(_NBUF):
                i = j * _NBUF + b
                g_copy(i, bank, b).wait()
                s_copy(i, bank, b).start()
        for bank in range(2):
            j = ngrp - 2 + bank
            for b in range(_NBUF):
                s_copy(j * _NBUF + b, bank, b).wait()

    return gather_kernel


def kernel(input, weight):
    B, H = input.shape
    V, D = weight.shape
    N = B * H
    idx = input.reshape(N // _CH, _CH).astype(jnp.int32)
    out = _build_gather(V, D, N)(weight, idx)
    return out.reshape(B, H, D)
